# baseline (speedup 1.0000x reference)
import jax
import jax.numpy as jnp
from jax import lax
from jax.experimental import pallas as pl
from jax.experimental.pallas import tpu as pltpu
from jax.experimental.pallas import tpu_sc as plsc

_S = 200
_B = 4096
_D = 64
_C = 512
_SUB = 128
_NSUB = _C // _SUB
_N = _S * _B
_NCHUNK = _N // _C
_CPS = _B // _C
_NC = 2
_NS = 16
_NW = _NC * _NS
_PER_W = _NCHUNK // _NW
_L = 16
_NQ = _D // _L
_SLAB = _D * _B
_TILEW = _SUB * 8
_CHW = _C * _D


def _emb_body(x_hbm, table_hbm, pos_hbm, out_hbm, idx_v, rows_v, outt_v, pos_v, gsem):
    wid = lax.axis_index("s") * _NC + lax.axis_index("c")
    pltpu.sync_copy(pos_hbm.at[pl.ds(0, _S)], pos_v)

    lane = lax.iota(jnp.int32, _L)
    cvq = [
        (2 * q + lane // 8) * (_NSUB * _TILEW) + (lane % 8) * _SUB
        for q in range(_NQ)
    ]

    def chunk_body(t, carry):
        g = wid * _PER_W + t
        pltpu.sync_copy(x_hbm.at[pl.ds(g * _NSUB, _NSUB)], idx_v)
        copies = [
            pltpu.async_copy(
                table_hbm.at[idx_v.at[j]],
                rows_v.at[pl.ds(j * _SUB, _SUB)],
                gsem,
            )
            for j in range(_NSUB)
        ]
        for cp in copies:
            cp.wait()

        s_idx = g // _CPS
        pos_regs = [pos_v[s_idx, pl.ds(q * _L, _L)] for q in range(_NQ)]

        def tok_body(tt, c2):
            off = (tt >> 7) * _TILEW + (tt & (_SUB - 1))
            for q in range(_NQ):
                src = rows_v[tt, pl.ds(q * _L, _L)] + pos_regs[q]
                plsc.store_scatter(outt_v, [cvq[q] + off], src)
            return c2

        lax.fori_loop(0, _C, tok_body, 0, unroll=4)

        j0 = g % _CPS
        for i in range(8):
            pltpu.sync_copy(
                outt_v.at[pl.ds(i * (_NSUB * _TILEW), _NSUB * _TILEW)],
                out_hbm.at[s_idx, pl.ds(i * (_CPS * _NSUB * _TILEW) + j0 * (_NSUB * _TILEW), _NSUB * _TILEW)],
            )
        return carry

    lax.fori_loop(0, _PER_W, chunk_body, 0)


@jax.jit
def _run(x, embed_table, pos_table):
    mesh = plsc.VectorSubcoreMesh(core_axis_name="c", subcore_axis_name="s")
    x2d = x.reshape(_N // _SUB, _SUB)
    out = pl.kernel(
        _emb_body,
        out_type=jax.ShapeDtypeStruct((_S, _SLAB), jnp.float32),
        mesh=mesh,
        scratch_types=[
            pltpu.VMEM((_NSUB, _SUB), jnp.int32),
            pltpu.VMEM((_C, _D), jnp.float32),
            pltpu.VMEM((_CHW,), jnp.float32),
            pltpu.VMEM((_S, _D), jnp.float32),
            pltpu.SemaphoreType.DMA,
        ],
        compiler_params=pltpu.CompilerParams(
            use_tc_tiling_on_sc=False, needs_layout_passes=False
        ),
    )(x2d, embed_table, pos_table)
    out = out.reshape(_S, 8, _B // _SUB, 8, _SUB)
    out = out.transpose(0, 1, 3, 2, 4).reshape(_S, _D, _B).transpose(0, 2, 1)
    return out


def kernel(x, embed_table, pos_table):
    return _run(x, embed_table, pos_table)


# speedup vs baseline: 1.5982x; 1.5982x over previous
import jax
import jax.numpy as jnp
from jax import lax
from jax.experimental import pallas as pl
from jax.experimental.pallas import tpu as pltpu
from jax.experimental.pallas import tpu_sc as plsc

_S = 200
_B = 4096
_D = 64
_C = 512
_SUB = 128
_NSUB = _C // _SUB
_N = _S * _B
_NCHUNK = _N // _C
_CPS = _B // _C
_NC = 2
_NS = 16
_NW = _NC * _NS
_PER_W = _NCHUNK // _NW
_L = 16
_NQ = _D // _L


def _emb_body(x_hbm, table_hbm, pos_hbm, out_hbm, idx_v, rows_v, pos_v, gsem, osem):
    wid = lax.axis_index("s") * _NC + lax.axis_index("c")
    pltpu.sync_copy(pos_hbm.at[pl.ds(0, _S)], pos_v)

    def stage_and_fire(t, buf):
        g = wid * _PER_W + t
        pltpu.sync_copy(x_hbm.at[pl.ds(g * _NSUB, _NSUB)], idx_v.at[buf])
        for j in range(_NSUB):
            pltpu.async_copy(
                table_hbm.at[idx_v.at[buf, j]],
                rows_v.at[buf, pl.ds(j * _SUB, _SUB)],
                gsem,
            )

    def drain_gather(buf):
        for j in range(_NSUB):
            pltpu.make_async_copy(
                table_hbm.at[idx_v.at[buf, j]],
                rows_v.at[buf, pl.ds(j * _SUB, _SUB)],
                gsem,
            ).wait()

    stage_and_fire(0, 0)

    def chunk_body(t, carry):
        g = wid * _PER_W + t
        buf = t % 2

        @pl.when(t + 1 < _PER_W)
        def _():
            stage_and_fire(t + 1, (t + 1) % 2)

        drain_gather(buf)

        s_idx = g // _CPS
        pos_regs = [pos_v[s_idx, pl.ds(q * _L, _L)] for q in range(_NQ)]

        def row_body(i, c2):
            for q in range(_NQ):
                plsc.addupdate(rows_v.at[buf, i, pl.ds(q * _L, _L)], pos_regs[q])
            return c2

        lax.fori_loop(0, _C, row_body, 0, unroll=8)

        @pl.when(t >= 2)
        def _():
            pltpu.make_async_copy(
                rows_v.at[buf],
                out_hbm.at[pl.ds((g - 2) * _C, _C)],
                osem,
            ).wait()

        pltpu.async_copy(rows_v.at[buf], out_hbm.at[pl.ds(g * _C, _C)], osem)
        return carry

    lax.fori_loop(0, _PER_W, chunk_body, 0)

    for tail in (_PER_W - 2, _PER_W - 1):
        g = wid * _PER_W + tail
        pltpu.make_async_copy(
            rows_v.at[tail % 2],
            out_hbm.at[pl.ds(g * _C, _C)],
            osem,
        ).wait()


@jax.jit
def _run(x, embed_table, pos_table):
    mesh = plsc.VectorSubcoreMesh(core_axis_name="c", subcore_axis_name="s")
    x2d = x.reshape(_N // _SUB, _SUB)
    tbl = lax.optimization_barrier(embed_table.reshape(500000, 128))
    tbl = tbl.reshape(1000000, _D)
    out = pl.kernel(
        _emb_body,
        out_type=jax.ShapeDtypeStruct((_N, _D), jnp.float32),
        mesh=mesh,
        scratch_types=[
            pltpu.VMEM((2, _NSUB, _SUB), jnp.int32),
            pltpu.VMEM((2, _C, _D), jnp.float32),
            pltpu.VMEM((_S, _D), jnp.float32),
            pltpu.SemaphoreType.DMA,
            pltpu.SemaphoreType.DMA,
        ],
        compiler_params=pltpu.CompilerParams(
            use_tc_tiling_on_sc=False, needs_layout_passes=False
        ),
    )(x2d, tbl, pos_table)
    return out.reshape(_S, _B, _D)


def kernel(x, embed_table, pos_table):
    return _run(x, embed_table, pos_table)
